# Initial kernel scaffold; baseline (speedup 1.0000x reference)
import jax
import jax.numpy as jnp
from jax.experimental import pallas as pl


def kernel(cond_mask, embed_weight):
    raise NotImplementedError("write your pallas kernel here")


# speedup vs baseline: 3.5976x; 3.5976x over previous
import math

import jax
import jax.numpy as jnp
from jax.experimental import pallas as pl

TIME_STEPS = 256
NUM_NODES = 128
EMBED_DIM = 16
CHANNELS = 128 + EMBED_DIM


def _body(wt_ref, out_ref):
    ci = jax.lax.broadcasted_iota(jnp.int32, (128, TIME_STEPS), 0)
    li = jax.lax.broadcasted_iota(jnp.int32, (128, TIME_STEPS), 1).astype(jnp.float32)
    half = (ci >> 1).astype(jnp.float32)
    inv_freq = jnp.exp(half * (-2.0 * math.log(10000.0) / 128.0))
    ang = li * inv_freq
    pe = jnp.where((ci & 1) == 0, jnp.sin(ang), jnp.cos(ang))
    out_ref[0, :128] = jnp.broadcast_to(pe[:, None, :], (128, NUM_NODES, TIME_STEPS))
    wt = wt_ref[...]
    out_ref[0, 128:] = jnp.broadcast_to(
        wt[:, :, None], (EMBED_DIM, NUM_NODES, TIME_STEPS)
    )


def kernel(cond_mask, embed_weight):
    B = cond_mask.shape[0]
    wt = embed_weight.T
    return pl.pallas_call(
        _body,
        grid=(B,),
        in_specs=[
            pl.BlockSpec((EMBED_DIM, NUM_NODES), lambda b: (0, 0)),
        ],
        out_specs=pl.BlockSpec(
            (1, CHANNELS, NUM_NODES, TIME_STEPS), lambda b: (b, 0, 0, 0)
        ),
        out_shape=jax.ShapeDtypeStruct(
            (B, CHANNELS, NUM_NODES, TIME_STEPS), jnp.float32
        ),
    )(wt)
